# baseline (speedup 1.0000x reference)
import functools

import jax
import jax.numpy as jnp
from jax import lax
from jax.experimental import pallas as pl
from jax.experimental.pallas import tpu as pltpu
from jax.experimental.pallas import tpu_sc as plsc

VOCAB = 128100
EMB = 512
HID = 1024
B = 4
S = 2048
EPS = 1e-07

N_TOK = B * S

_CHUNK = 128


def _make_sc_gather():
    info = plsc.get_sparse_core_info()
    nc, ns = info.num_cores, info.num_subcores
    nw = nc * ns
    per_w = N_TOK // nw
    n_chunks = per_w // _CHUNK
    mesh = plsc.VectorSubcoreMesh(core_axis_name="c", subcore_axis_name="s")

    @functools.partial(
        pl.kernel,
        mesh=mesh,
        out_type=jax.ShapeDtypeStruct((N_TOK, EMB), jnp.float32),
        scratch_types=[
            pltpu.VMEM((_CHUNK,), jnp.int32),
            pltpu.VMEM((_CHUNK, EMB), jnp.float32),
            pltpu.SemaphoreType.DMA,
        ],
    )
    def gather_k(idx_hbm, table_hbm, out_hbm, idx_v, rows_v, sem):
        wid = lax.axis_index("s") * nc + lax.axis_index("c")
        w_per_row = S // per_w
        row = wid // w_per_row
        col0 = (wid % w_per_row) * per_w
        base0 = wid * per_w
        for c in range(n_chunks):
            pltpu.sync_copy(idx_hbm.at[row, pl.ds(col0 + c * _CHUNK, _CHUNK)],
                            idx_v)
            pltpu.async_copy(table_hbm.at[idx_v], rows_v, sem).wait()
            pltpu.sync_copy(rows_v,
                            out_hbm.at[pl.ds(base0 + c * _CHUNK, _CHUNK)])

    return gather_k


_BLK = 2048


def _tc_body(g_ref, p_ref, w_ref, gamma_ref, beta_ref, o_ref):
    x = (g_ref[...] + p_ref[...]).astype(jnp.bfloat16)
    y = lax.dot_general(x, w_ref[...].astype(jnp.bfloat16),
                        (((1,), (1,)), ((), ())),
                        preferred_element_type=jnp.float32)
    mean = jnp.mean(y, axis=-1, keepdims=True)
    yc = y - mean
    var = jnp.mean(yc * yc, axis=-1, keepdims=True)
    o_ref[0] = yc * lax.rsqrt(var + EPS) * gamma_ref[...] + beta_ref[...]


def _tc_call(gathered, pos, w, gamma, beta):
    s_blocks = S // _BLK
    return pl.pallas_call(
        _tc_body,
        grid=(s_blocks, B),
        in_specs=[
            pl.BlockSpec((_BLK, EMB), lambda i, j: (j * s_blocks + i, 0)),
            pl.BlockSpec((_BLK, EMB), lambda i, j: (i, 0)),
            pl.BlockSpec((HID, EMB), lambda i, j: (0, 0)),
            pl.BlockSpec((1, HID), lambda i, j: (0, 0)),
            pl.BlockSpec((1, HID), lambda i, j: (0, 0)),
        ],
        out_specs=pl.BlockSpec((1, _BLK, HID),
                               lambda i, j: (j * s_blocks + i, 0, 0)),
        out_shape=jax.ShapeDtypeStruct((B, S, HID), jnp.float32),
    )(gathered, pos, w, gamma, beta)


def kernel(input_ids, word_embeddings, position_embeddings, proj_weight, ln_gamma, ln_beta):
    gathered = _make_sc_gather()(input_ids, word_embeddings)
    return _tc_call(
        gathered,
        position_embeddings,
        proj_weight,
        ln_gamma.reshape(1, HID),
        ln_beta.reshape(1, HID),
    )
